# baseline (speedup 1.0000x reference)
import jax
import jax.numpy as jnp
from jax import lax
from jax.experimental import pallas as pl
from jax.experimental.pallas import tpu as pltpu

N = 16384
D = 256
K = 8192
BN = 512
NB = N // BN
CHUNK = 2736
POS_INF = float("inf")
ONE_BITS = 0x3F800000


def _vq_body(x_ref, emb_ref, out_ref, mm2_ref, xx_ref, ee_ref, e2_ref):
    i = pl.program_id(0)

    @pl.when(i == 0)
    def _():
        eb = emb_ref[...]
        ee_ref[...] = jnp.sum(eb * eb, axis=1, keepdims=True)
        e2_ref[...] = (eb + eb).astype(jnp.bfloat16)

    par = lax.rem(i, 2)

    xb = x_ref[...]
    e2 = e2_ref[...]
    mm2_ref[pl.ds(par * K, K), :] = lax.dot_general(
        e2, xb.astype(jnp.bfloat16),
        (((1,), (1,)), ((), ())),
        preferred_element_type=jnp.float32,
    )
    xx = jnp.sum(xb * xb, axis=1, keepdims=True)
    xx_ref[pl.ds(par, 1), :] = lax.transpose(xx, (1, 0))

    ppar = lax.rem(i + 1, 2)
    base = ppar * K
    xx_t = xx_ref[pl.ds(ppar, 1), :]
    acc_v = jnp.full((BN,), POS_INF, jnp.float32)
    acc_i = jnp.full((BN,), POS_INF, jnp.float32)
    base_iota = lax.broadcasted_iota(jnp.int32, (8, BN), 0)
    NACC = 4
    for c0 in range(0, K, CHUNK):
        hi = min(c0 + CHUNK, K)
        rv = [jnp.full((8, BN), POS_INF, jnp.float32) for _ in range(NACC)]
        ri = [jnp.full((8, BN), POS_INF, jnp.float32) for _ in range(NACC)]
        for j, s in enumerate(range(c0, hi, 8)):
            g = j % NACC
            ms = mm2_ref[pl.ds(base + s, 8), :]
            ts = (xx_t - ms) + ee_ref[pl.ds(s, 8), :]
            enc = lax.bitcast_convert_type(
                base_iota + jnp.int32(ONE_BITS + s), jnp.float32)
            lt = ts < rv[g]
            rv[g] = jnp.where(lt, ts, rv[g])
            ri[g] = jnp.where(lt, enc, ri[g])
        mv = rv[0]
        for g in range(1, NACC):
            mv = jnp.minimum(mv, rv[g])
        mi = jnp.full((8, BN), POS_INF, jnp.float32)
        for g in range(NACC):
            mi = jnp.minimum(mi, jnp.where(rv[g] == mv, ri[g], POS_INF))
        lm = jnp.min(mv, axis=0)
        li = jnp.min(jnp.where(mv == lm[None, :], mi, POS_INF), axis=0)
        take = lm < acc_v
        acc_i = jnp.where(take, li, acc_i)
        acc_v = jnp.where(take, lm, acc_v)
        acc_v = acc_v.astype(jnp.bfloat16).astype(jnp.float32)
    out_ref[...] = (lax.bitcast_convert_type(acc_i, jnp.int32)
                    - jnp.int32(ONE_BITS))[None, None, :]


def kernel(x, inited, cluster_size, embed, embed_avg):
    del inited, cluster_size, embed_avg
    xf = x.reshape(N, D)
    out = pl.pallas_call(
        _vq_body,
        grid=(NB + 1,),
        in_specs=[
            pl.BlockSpec((BN, D), lambda i: (jnp.minimum(i, NB - 1), 0)),
            pl.BlockSpec((K, D), lambda i: (0, 0)),
        ],
        out_specs=pl.BlockSpec(
            (1, 1, BN), lambda i: (jnp.maximum(i - 1, 0), 0, 0)),
        out_shape=jax.ShapeDtypeStruct((NB, 1, BN), jnp.int32),
        scratch_shapes=[
            pltpu.VMEM((2 * K, BN), jnp.float32),
            pltpu.VMEM((2, BN), jnp.float32),
            pltpu.VMEM((K, 1), jnp.float32),
            pltpu.VMEM((K, D), jnp.bfloat16),
        ],
    )(xf, embed)
    return out.reshape(x.shape[:-1])


# speedup vs baseline: 1.5995x; 1.5995x over previous
import jax
import jax.numpy as jnp
from jax import lax
from jax.experimental import pallas as pl
from jax.experimental.pallas import tpu as pltpu

N = 16384
D = 256
K = 8192
BN = 512
NB = N // BN
CHUNK = 2736
POS_INF = float("inf")
ONE_BITS = 0x3F800000


def _vq_body(x_ref, emb_ref, out_ref, mm2_ref, ee_ref, e2_ref):
    @pl.when(pl.program_id(0) == 0)
    def _():
        eb = emb_ref[...]
        ee_ref[...] = jnp.sum(eb * eb, axis=1, keepdims=True)
        e2_ref[...] = (eb + eb).astype(jnp.bfloat16)

    xb = x_ref[...]
    e2 = e2_ref[...]
    mm2_ref[...] = lax.dot_general(
        e2, xb.astype(jnp.bfloat16),
        (((1,), (1,)), ((), ())),
        preferred_element_type=jnp.float32,
    )
    xx = jnp.sum(xb * xb, axis=1, keepdims=True)
    xx_t = lax.transpose(xx, (1, 0))
    acc_v = jnp.full((BN,), POS_INF, jnp.float32)
    acc_i = jnp.full((BN,), POS_INF, jnp.float32)
    base_iota = lax.broadcasted_iota(jnp.int32, (8, BN), 0)
    NACC = 4
    for c0 in range(0, K, CHUNK):
        hi = min(c0 + CHUNK, K)
        rv = [jnp.full((8, BN), POS_INF, jnp.float32) for _ in range(NACC)]
        ri = [jnp.full((8, BN), POS_INF, jnp.float32) for _ in range(NACC)]
        for j, s in enumerate(range(c0, hi, 8)):
            g = j % NACC
            ms = mm2_ref[pl.ds(s, 8), :]
            ts = (xx_t - ms) + ee_ref[pl.ds(s, 8), :]
            enc = lax.bitcast_convert_type(
                base_iota + jnp.int32(ONE_BITS + s), jnp.float32)
            lt = ts < rv[g]
            rv[g] = jnp.where(lt, ts, rv[g])
            ri[g] = jnp.where(lt, enc, ri[g])
        mv = rv[0]
        for g in range(1, NACC):
            mv = jnp.minimum(mv, rv[g])
        mi = jnp.full((8, BN), POS_INF, jnp.float32)
        for g in range(NACC):
            mi = jnp.minimum(mi, jnp.where(rv[g] == mv, ri[g], POS_INF))
        lm = jnp.min(mv, axis=0)
        li = jnp.min(jnp.where(mv == lm[None, :], mi, POS_INF), axis=0)
        take = lm < acc_v
        acc_i = jnp.where(take, li, acc_i)
        acc_v = jnp.where(take, lm, acc_v)
        acc_v = acc_v.astype(jnp.bfloat16).astype(jnp.float32)
    out_ref[...] = (lax.bitcast_convert_type(acc_i, jnp.int32)
                    - jnp.int32(ONE_BITS))[None, None, :]


def kernel(x, inited, cluster_size, embed, embed_avg):
    del inited, cluster_size, embed_avg
    xf = x.reshape(N, D)
    out = pl.pallas_call(
        _vq_body,
        grid=(NB,),
        in_specs=[
            pl.BlockSpec((BN, D), lambda i: (i, 0)),
            pl.BlockSpec((K, D), lambda i: (0, 0)),
        ],
        out_specs=pl.BlockSpec((1, 1, BN), lambda i: (i, 0, 0)),
        out_shape=jax.ShapeDtypeStruct((NB, 1, BN), jnp.int32),
        scratch_shapes=[
            pltpu.VMEM((K, BN), jnp.float32),
            pltpu.VMEM((K, 1), jnp.float32),
            pltpu.VMEM((K, D), jnp.bfloat16),
        ],
    )(xf, embed)
    return out.reshape(x.shape[:-1])
